# baseline (speedup 1.0000x reference)
import functools

import jax
import jax.numpy as jnp
import numpy as np
from jax import lax
from jax.experimental import pallas as pl
from jax.experimental.pallas import tpu as pltpu
from jax.experimental.pallas import tpu_sc as plsc

_BATCH = 65536
_N_PRED = 256
_COLS = (0, 3, 17, 42, 97, 128, 200, 255)
_SIGNS = (-1.0, 1.0, -1.0, 1.0, 1.0, -1.0, 1.0, -1.0)
_L = len(_COLS)
_MIN_W = 0.0
_MAX_W = 500.0

_LANES = 16
_NUM_CORES = 2
_NUM_SUBCORES = 16
_NW = _NUM_CORES * _NUM_SUBCORES
_RPW = _BATCH // _NW
_WPW = _RPW * _L
_STEPS = _RPW // _LANES

_IDX_CONST = np.asarray(_COLS, dtype=np.int32).reshape(-1, 1)

_ROWS_NP = np.arange(_BATCH, dtype=np.int64)
_COLS_NP = np.asarray(_COLS, dtype=np.int64)
_PHYS = (
    (_ROWS_NP[:, None] >> 3) * 2048
    + (_COLS_NP[None, :] >> 7) * 1024
    + (_ROWS_NP[:, None] & 7) * 128
    + (_COLS_NP[None, :] & 127)
)
_GATHER_WORDS = _PHYS.reshape(-1).astype(np.int32)


_SLABS = 8
_SLAB_W = _WPW // _SLABS
_SLAB_STEPS = _SLAB_W // (_LANES * _L)


def _tec_body(ga_hbm, idx_hbm, w_hbm, out_hbm, idxv, colv, outv, wv, *sems):
    wid = lax.axis_index("s") * _NUM_CORES + lax.axis_index("c")
    base = wid * _WPW

    pltpu.sync_copy(idx_hbm.at[pl.ds(base, _WPW)], idxv)
    gathers = []
    for s in range(_SLABS):
        cp = pltpu.make_async_copy(
            ga_hbm.at[idxv.at[pl.ds(s * _SLAB_W, _SLAB_W)]],
            colv.at[pl.ds(s * _SLAB_W, _SLAB_W)], sems[s])
        cp.start()
        gathers.append(cp)

    pltpu.sync_copy(w_hbm, wv)
    w16 = wv[...]
    w16 = jnp.minimum(jnp.maximum(w16, _MIN_W), _MAX_W)

    lane8 = lax.broadcasted_iota(jnp.int32, (_LANES,), 0) * _L

    def step(i, carry):
        pidx = i * (_LANES * _L) + lane8
        xs = []
        for j, sg in enumerate(_SIGNS):
            x = plsc.load_gather(colv, [pidx + j])
            xs.append(-x if sg < 0 else x)
        m = xs[0]
        for x in xs[1:]:
            m = jnp.maximum(m, x)
        es = [jnp.exp(x - m) for x in xs]
        tot = es[0]
        for e in es[1:]:
            tot = tot + e
        scale = w16 / tot
        off = (i >> 3) * 1024 + (i & 7) * _LANES
        for j, sg in enumerate(_SIGNS):
            d = es[j] * scale
            if sg < 0:
                d = -d
            outv[pl.ds(off + j * 128, _LANES)] = d
        return carry

    for s in range(_SLABS):
        gathers[s].wait()
        lax.fori_loop(s * _SLAB_STEPS, (s + 1) * _SLAB_STEPS, step, 0)

    pltpu.sync_copy(outv, out_hbm.at[pl.ds(base, _WPW)])


@jax.jit
def _delta_sc(ga_lin, gather_words, wvec):
    mesh = plsc.VectorSubcoreMesh(core_axis_name="c", subcore_axis_name="s")
    k = functools.partial(
        pl.kernel,
        mesh=mesh,
        compiler_params=pltpu.CompilerParams(
            use_tc_tiling_on_sc=False, needs_layout_passes=False),
        out_type=jax.ShapeDtypeStruct((_BATCH * _L,), jnp.float32),
        scratch_types=[
            pltpu.VMEM((_WPW,), jnp.int32),
            pltpu.VMEM((_WPW,), jnp.float32),
            pltpu.VMEM((_WPW,), jnp.float32),
            pltpu.VMEM((_LANES,), jnp.float32),
        ] + [pltpu.SemaphoreType.DMA] * _SLABS,
    )(_tec_body)
    return k(ga_lin, gather_words, wvec)


def kernel(ground_atoms, clause_weight):
    wvec = jnp.broadcast_to(jnp.reshape(clause_weight, (1,)), (_LANES,))
    ga_lin = (
        ground_atoms.reshape(_BATCH // 8, 8, _N_PRED // 128, 128)
        .transpose(0, 2, 1, 3)
        .reshape(-1)
    )
    flat = _delta_sc(ga_lin, jnp.asarray(_GATHER_WORDS), wvec)
    delta = (
        flat.reshape(_BATCH // 128, _L, 128)
        .transpose(0, 2, 1)
        .reshape(_BATCH, _L)
    )
    return (delta, jnp.asarray(_IDX_CONST))


# speedup vs baseline: 1.1395x; 1.1395x over previous
import functools

import jax
import jax.numpy as jnp
import numpy as np
from jax import lax
from jax.experimental import pallas as pl
from jax.experimental.pallas import tpu as pltpu
from jax.experimental.pallas import tpu_sc as plsc

_BATCH = 65536
_N_PRED = 256
_COLS = (0, 3, 17, 42, 97, 128, 200, 255)
_SIGNS = (-1.0, 1.0, -1.0, 1.0, 1.0, -1.0, 1.0, -1.0)
_L = len(_COLS)
_MIN_W = 0.0
_MAX_W = 500.0

_LANES = 16
_NUM_CORES = 2
_NUM_SUBCORES = 16
_NW = _NUM_CORES * _NUM_SUBCORES
_RPW = _BATCH // _NW
_WPW = _RPW * _L
_CHUNK = 128
_CHUNK_W = _CHUNK * _N_PRED
_NCHUNK = _RPW // _CHUNK
_GROUPS = _CHUNK // _LANES

_IDX_CONST = np.asarray(_COLS, dtype=np.int32).reshape(-1, 1)

_COLTERM = tuple(((c >> 7) * 1024 + (c & 127)) for c in _COLS)


def _tec_body(ga_hbm, w_hbm, out_hbm, sa, sb, outv, wv, sema, semb):
    wid = lax.axis_index("s") * _NUM_CORES + lax.axis_index("c")
    wbase = wid * (_RPW * _N_PRED)

    pltpu.sync_copy(w_hbm, wv)
    w16 = wv[...]
    w16 = jnp.minimum(jnp.maximum(w16, _MIN_W), _MAX_W)

    lane = lax.broadcasted_iota(jnp.int32, (_LANES,), 0)
    laneoff = (lane >> 3) * 2048 + (lane & 7) * 128

    pltpu.make_async_copy(
        ga_hbm.at[pl.ds(wbase, _CHUNK_W)], sa, sema).start()
    pltpu.make_async_copy(
        ga_hbm.at[pl.ds(wbase + _CHUNK_W, _CHUNK_W)], sb, semb).start()

    def body(g, carry):
        for b, (buf, sem) in enumerate(((sa, sema), (sb, semb))):
            c = 2 * g + b
            slab0 = wbase + c * _CHUNK_W
            pltpu.make_async_copy(
                ga_hbm.at[pl.ds(slab0, _CHUNK_W)], buf, sem).wait()

            for s in range(_GROUPS):
                ivec = laneoff + (s * 4096)
                xs = []
                for j, sg in enumerate(_SIGNS):
                    x = plsc.load_gather(buf, [ivec + _COLTERM[j]])
                    xs.append(-x if sg < 0 else x)
                m = xs[0]
                for x in xs[1:]:
                    m = jnp.maximum(m, x)
                es = [jnp.exp(x - m) for x in xs]
                tot = es[0]
                for e in es[1:]:
                    tot = tot + e
                scale = w16 / tot
                off = c * 1024 + s * _LANES
                for j, sg in enumerate(_SIGNS):
                    d = es[j] * scale
                    if sg < 0:
                        d = -d
                    outv[pl.ds(off + j * 128, _LANES)] = d

            @pl.when(g < _NCHUNK // 2 - 1)
            def _prefetch():
                pltpu.make_async_copy(
                    ga_hbm.at[pl.ds(slab0 + 2 * _CHUNK_W, _CHUNK_W)], buf,
                    sem).start()
        return carry

    lax.fori_loop(0, _NCHUNK // 2, body, 0)
    pltpu.sync_copy(outv, out_hbm.at[pl.ds(wid * _WPW, _WPW)])


@jax.jit
def _delta_sc(ga_lin, wvec):
    mesh = plsc.VectorSubcoreMesh(core_axis_name="c", subcore_axis_name="s")
    k = functools.partial(
        pl.kernel,
        mesh=mesh,
        compiler_params=pltpu.CompilerParams(
            use_tc_tiling_on_sc=False, needs_layout_passes=False),
        out_type=jax.ShapeDtypeStruct((_BATCH * _L,), jnp.float32),
        scratch_types=[
            pltpu.VMEM((_CHUNK_W,), jnp.float32),
            pltpu.VMEM((_CHUNK_W,), jnp.float32),
            pltpu.VMEM((_WPW,), jnp.float32),
            pltpu.VMEM((_LANES,), jnp.float32),
            pltpu.SemaphoreType.DMA,
            pltpu.SemaphoreType.DMA,
        ],
    )(_tec_body)
    return k(ga_lin, wvec)


def kernel(ground_atoms, clause_weight):
    wvec = jnp.broadcast_to(jnp.reshape(clause_weight, (1,)), (_LANES,))
    ga_lin = (
        ground_atoms.reshape(_BATCH // 8, 8, _N_PRED // 128, 128)
        .transpose(0, 2, 1, 3)
        .reshape(-1)
    )
    flat = _delta_sc(ga_lin, wvec)
    delta = (
        flat.reshape(_BATCH // 128, _L, 128)
        .transpose(0, 2, 1)
        .reshape(_BATCH, _L)
    )
    return (delta, jnp.asarray(_IDX_CONST))
